# Initial kernel scaffold; baseline (speedup 1.0000x reference)
import jax
import jax.numpy as jnp
from jax.experimental import pallas as pl


def kernel(x, W1, b1, W2):
    raise NotImplementedError("write your pallas kernel here")


# speedup vs baseline: 1.4390x; 1.4390x over previous
import functools

import jax
import jax.numpy as jnp
from jax.experimental import pallas as pl
from jax.experimental.pallas import tpu as pltpu


def _gate_kernel(x_ref, w1_ref, b1_ref, w2_ref,
                 scores_ref, idx_ref, loss_ref, acc_ref,
                 *, total_tokens: int, num_experts: int):
    i = pl.program_id(0)
    n = pl.num_programs(0)

    x = x_ref[...]
    h = jnp.maximum(
        jax.lax.dot_general(x, w1_ref[...], (((1,), (0,)), ((), ())),
                            preferred_element_type=jnp.float32)
        + b1_ref[...],
        0.0)
    logits = jax.lax.dot_general(h, w2_ref[...], (((1,), (0,)), ((), ())),
                                 preferred_element_type=jnp.float32)

    row_max = jnp.max(logits, axis=-1, keepdims=True)
    ex = jnp.exp(logits - row_max)
    scores = ex / jnp.sum(ex, axis=-1, keepdims=True)

    iota = jax.lax.broadcasted_iota(jnp.int32, scores.shape, 1)
    m1 = jnp.max(scores, axis=-1, keepdims=True)
    a1 = jnp.min(jnp.where(scores == m1, iota, num_experts), axis=-1,
                 keepdims=True)
    masked = jnp.where(iota == a1, -jnp.inf, scores)
    m2 = jnp.max(masked, axis=-1, keepdims=True)
    a2 = jnp.min(jnp.where(masked == m2, iota, num_experts), axis=-1,
                 keepdims=True)

    t = jnp.exp(m2 - m1)
    denom = 1.0 + t
    scores_ref[...] = jnp.concatenate([1.0 / denom, t / denom], axis=-1)
    idx_ref[...] = jnp.concatenate([a1, a2], axis=-1)

    part = jnp.sum(scores, axis=0, keepdims=True)

    @pl.when(i == 0)
    def _():
        acc_ref[...] = part

    @pl.when(i > 0)
    def _():
        acc_ref[...] += part

    @pl.when(i == n - 1)
    def _():
        mean = acc_ref[...] / total_tokens
        loss_ref[...] = num_experts * jnp.sum(mean * jnp.log(mean + 1e-8),
                                              axis=-1, keepdims=True)


def kernel(x, W1, b1, W2):
    batch, seq, d_model = x.shape
    m = batch * seq
    d_hidden = W1.shape[1]
    num_experts = W2.shape[1]

    tm = min(256, m)
    grid = (m // tm,)

    x_flat = x.reshape(m, d_model)
    b1_2d = b1.reshape(1, d_hidden)

    scores, idx, loss = pl.pallas_call(
        functools.partial(_gate_kernel, total_tokens=m,
                          num_experts=num_experts),
        grid=grid,
        in_specs=[
            pl.BlockSpec((tm, d_model), lambda i: (i, 0)),
            pl.BlockSpec((d_model, d_hidden), lambda i: (0, 0)),
            pl.BlockSpec((1, d_hidden), lambda i: (0, 0)),
            pl.BlockSpec((d_hidden, num_experts), lambda i: (0, 0)),
        ],
        out_specs=[
            pl.BlockSpec((tm, 2), lambda i: (i, 0)),
            pl.BlockSpec((tm, 2), lambda i: (i, 0)),
            pl.BlockSpec((1, 1), lambda i: (0, 0)),
        ],
        out_shape=[
            jax.ShapeDtypeStruct((m, 2), jnp.float32),
            jax.ShapeDtypeStruct((m, 2), jnp.int32),
            jax.ShapeDtypeStruct((1, 1), jnp.float32),
        ],
        scratch_shapes=[pltpu.VMEM((1, num_experts), jnp.float32)],
    )(x_flat, W1, b1_2d, W2)

    return scores, idx, loss[0, 0]


# speedup vs baseline: 1.5171x; 1.0543x over previous
import functools

import jax
import jax.numpy as jnp
from jax.experimental import pallas as pl
from jax.experimental.pallas import tpu as pltpu


def _gate_kernel(x_ref, w1_ref, b1_ref, w2_ref,
                 scores_ref, idx_ref, loss_ref, acc_ref,
                 *, total_tokens: int, num_experts: int):
    i = pl.program_id(0)
    n = pl.num_programs(0)

    x = x_ref[...]
    h = jnp.maximum(
        jax.lax.dot_general(x, w1_ref[...], (((1,), (0,)), ((), ())),
                            preferred_element_type=jnp.float32)
        + b1_ref[...],
        0.0)
    logits = jax.lax.dot_general(h, w2_ref[...], (((1,), (0,)), ((), ())),
                                 preferred_element_type=jnp.float32)

    row_max = jnp.max(logits, axis=-1, keepdims=True)
    ex = jnp.exp(logits - row_max)
    scores = ex / jnp.sum(ex, axis=-1, keepdims=True)

    iota = jax.lax.broadcasted_iota(jnp.int32, scores.shape, 1)
    m1 = jnp.max(scores, axis=-1, keepdims=True)
    a1 = jnp.min(jnp.where(scores == m1, iota, num_experts), axis=-1,
                 keepdims=True)
    masked = jnp.where(iota == a1, -jnp.inf, scores)
    m2 = jnp.max(masked, axis=-1, keepdims=True)
    a2 = jnp.min(jnp.where(masked == m2, iota, num_experts), axis=-1,
                 keepdims=True)

    t = jnp.exp(m2 - m1)
    denom = 1.0 + t
    scores_ref[...] = jnp.concatenate([1.0 / denom, t / denom], axis=-1)
    idx_ref[...] = jnp.concatenate([a1, a2], axis=-1)

    part = jnp.sum(scores, axis=0, keepdims=True)

    @pl.when(i == 0)
    def _():
        acc_ref[...] = part

    @pl.when(i > 0)
    def _():
        acc_ref[...] += part

    @pl.when(i == n - 1)
    def _():
        mean = acc_ref[...] / total_tokens
        loss_ref[...] = num_experts * jnp.sum(mean * jnp.log(mean + 1e-8),
                                              axis=-1, keepdims=True)


def kernel(x, W1, b1, W2):
    batch, seq, d_model = x.shape
    m = batch * seq
    d_hidden = W1.shape[1]
    num_experts = W2.shape[1]

    tm = min(512, m)
    grid = (m // tm,)

    x_flat = x.reshape(m, d_model)
    b1_2d = b1.reshape(1, d_hidden)

    scores, idx, loss = pl.pallas_call(
        functools.partial(_gate_kernel, total_tokens=m,
                          num_experts=num_experts),
        grid=grid,
        in_specs=[
            pl.BlockSpec((tm, d_model), lambda i: (i, 0)),
            pl.BlockSpec((d_model, d_hidden), lambda i: (0, 0)),
            pl.BlockSpec((1, d_hidden), lambda i: (0, 0)),
            pl.BlockSpec((d_hidden, num_experts), lambda i: (0, 0)),
        ],
        out_specs=[
            pl.BlockSpec((tm, 2), lambda i: (i, 0)),
            pl.BlockSpec((tm, 2), lambda i: (i, 0)),
            pl.BlockSpec((1, 1), lambda i: (0, 0)),
        ],
        out_shape=[
            jax.ShapeDtypeStruct((m, 2), jnp.float32),
            jax.ShapeDtypeStruct((m, 2), jnp.int32),
            jax.ShapeDtypeStruct((1, 1), jnp.float32),
        ],
        scratch_shapes=[pltpu.VMEM((1, num_experts), jnp.float32)],
    )(x_flat, W1, b1_2d, W2)

    return scores, idx, loss[0, 0]
